# baseline (speedup 1.0000x reference)
import functools

import jax
import jax.numpy as jnp
from jax import lax
from jax.experimental import pallas as pl
from jax.experimental.pallas import tpu as pltpu
from jax.experimental.pallas import tpu_sc as plsc

D_MODEL = 256
KNN = 16
N_PTS = 2048
N_BATCH = 4
BN = 512
BM = 512
N_WORKERS = 32
W_PER_B = N_WORKERS // N_BATCH
ROWS_PER_W = N_PTS // W_PER_B


def _topk_body(pc_ref, pcT_ref, idx_ref):
    n_blk = pl.program_id(1)
    q = pc_ref[0]
    qx = q[:, 0:1]
    qy = q[:, 1:2]
    qz = q[:, 2:3]
    xr = pcT_ref[0, 0:1, :]
    yr = pcT_ref[0, 1:2, :]
    zr = pcT_ref[0, 2:3, :]
    dx = qx - xr
    dy = qy - yr
    dz = qz - zr
    vals = dx * dx + dy * dy + dz * dz
    iota = lax.broadcasted_iota(jnp.int32, (BN, N_PTS), 1)
    row_ids = (
        lax.broadcasted_iota(jnp.int32, (BN, 1), 0) + n_blk * BN
    )
    vals = jnp.where(iota == row_ids, jnp.inf, vals)
    iota_i = lax.broadcasted_iota(jnp.int32, (N_PTS, 1), 0)
    iota_hi = (iota_i // 256).astype(jnp.float32)
    iota_lo = (iota_i % 256).astype(jnp.float32)
    iota_hl = jnp.concatenate([iota_hi, iota_lo], axis=1)
    cols = [row_ids.astype(jnp.float32)]
    for _ in range(KNN - 1):
        m = jnp.min(vals, axis=1, keepdims=True)
        eq = vals == m
        sel01 = jnp.where(eq, 1.0, 0.0)
        hl = jnp.dot(sel01, iota_hl, preferred_element_type=jnp.float32)
        vals = jnp.where(eq, jnp.inf, vals)
        cols.append(hl[:, 0:1] * 256.0 + hl[:, 1:2])
    idx = jnp.concatenate(cols, axis=1)
    idx_ref[0] = (idx + 0.5).astype(jnp.int32)


def _gather_body(pcT_hbm, idx_hbm, out_hbm, x_v, y_v, z_v, idx_v, gx_v, gy_v, gz_v):
    wid = lax.axis_index("s") * 2 + lax.axis_index("c")
    b = wid // W_PER_B
    r0 = (wid % W_PER_B) * ROWS_PER_W

    pltpu.sync_copy(pcT_hbm.at[pl.ds(b * 3 * N_PTS, N_PTS)], x_v)
    pltpu.sync_copy(pcT_hbm.at[pl.ds(b * 3 * N_PTS + N_PTS, N_PTS)], y_v)
    pltpu.sync_copy(pcT_hbm.at[pl.ds(b * 3 * N_PTS + 2 * N_PTS, N_PTS)], z_v)
    pltpu.sync_copy(
        idx_hbm.at[pl.ds((b * N_PTS + r0) * KNN, ROWS_PER_W * KNN)], idx_v)

    lane = lax.iota(jnp.int32, KNN)

    def body(i, carry):
        pos = lane + i * KNN
        iv = plsc.load_gather(idx_v, [pos])
        gx = plsc.load_gather(x_v, [iv])
        gy = plsc.load_gather(y_v, [iv])
        gz = plsc.load_gather(z_v, [iv])
        plsc.store_scatter(gx_v, [pos], gx)
        plsc.store_scatter(gy_v, [pos], gy)
        plsc.store_scatter(gz_v, [pos], gz)
        return carry

    lax.fori_loop(0, ROWS_PER_W, body, 0)

    base = (b * 3 * N_PTS + r0) * KNN
    pltpu.sync_copy(gx_v, out_hbm.at[pl.ds(base, ROWS_PER_W * KNN)])
    pltpu.sync_copy(gy_v, out_hbm.at[pl.ds(base + N_PTS * KNN, ROWS_PER_W * KNN)])
    pltpu.sync_copy(gz_v, out_hbm.at[pl.ds(base + 2 * N_PTS * KNN, ROWS_PER_W * KNN)])


def _mlp_body(pc_ref, g_ref, w1_ref, w1e_hi_ref, w1e_lo_ref, b1_ref, w2_ref,
              b2_ref, out_ref):
    q = pc_ref[0]
    qx = q[:, 0:1]
    qy = q[:, 1:2]
    qz = q[:, 2:3]
    w1x = w1_ref[0:1, :]
    w1y = w1_ref[1:2, :]
    w1z = w1_ref[2:3, :]
    b1 = b1_ref[...]
    w2 = w2_ref[...]
    b2 = b2_ref[...]
    qpart = qx * w1x + qy * w1y + qz * w1z + b1
    g3 = jnp.concatenate(
        [g_ref[0, 0], g_ref[0, 1], g_ref[0, 2]], axis=1)
    g3_hi = g3.astype(jnp.bfloat16).astype(jnp.float32)
    g3_lo = (g3 - g3_hi) * 512.0
    o0 = jnp.dot(jnp.maximum(b1, 0.0), w2, preferred_element_type=jnp.float32) + b2
    out_ref[0, :, 0, :] = jnp.broadcast_to(o0, (BM, D_MODEL))
    for k in range(1, KNN):
        whi = w1e_hi_ref[k * 48:(k + 1) * 48, :]
        wlo = w1e_lo_ref[k * 48:(k + 1) * 48, :]
        gpart = (jnp.dot(g3_hi, whi, preferred_element_type=jnp.float32)
                 + jnp.dot(g3_hi, wlo, preferred_element_type=jnp.float32)
                 * (1.0 / 512.0)
                 + jnp.dot(g3_lo, whi, preferred_element_type=jnp.float32)
                 * (1.0 / 512.0))
        h = jnp.maximum(qpart - gpart, 0.0)
        out_ref[0, :, k, :] = jnp.dot(h, w2, preferred_element_type=jnp.float32) + b2


def _sc_gather(pcT_flat, idx_flat):
    mesh = plsc.VectorSubcoreMesh(core_axis_name="c", subcore_axis_name="s")
    f = pl.kernel(
        _gather_body,
        out_type=jax.ShapeDtypeStruct((N_BATCH * 3 * N_PTS * KNN,), jnp.float32),
        mesh=mesh,
        compiler_params=pltpu.CompilerParams(needs_layout_passes=False),
        scratch_types=[
            pltpu.VMEM((N_PTS,), jnp.float32),
            pltpu.VMEM((N_PTS,), jnp.float32),
            pltpu.VMEM((N_PTS,), jnp.float32),
            pltpu.VMEM((ROWS_PER_W * KNN,), jnp.int32),
            pltpu.VMEM((ROWS_PER_W * KNN,), jnp.float32),
            pltpu.VMEM((ROWS_PER_W * KNN,), jnp.float32),
            pltpu.VMEM((ROWS_PER_W * KNN,), jnp.float32),
        ],
    )
    return f(pcT_flat, idx_flat)


@jax.jit
def kernel(pc, W1, b1, W2, b2):
    B, N, _ = pc.shape
    pcT = jnp.transpose(pc, (0, 2, 1))
    b1r = b1.reshape(1, D_MODEL)
    b2r = b2.reshape(1, D_MODEL)

    idx = pl.pallas_call(
        _topk_body,
        grid=(B, N // BN),
        in_specs=[
            pl.BlockSpec((1, BN, 3), lambda b, n: (b, n, 0)),
            pl.BlockSpec((1, 3, N_PTS), lambda b, n: (b, 0, 0)),
        ],
        out_specs=pl.BlockSpec((1, BN, KNN), lambda b, n: (b, n, 0)),
        out_shape=jax.ShapeDtypeStruct((B, N, KNN), jnp.int32),
    )(pc, pcT)

    g_flat = _sc_gather(pcT.reshape(-1), idx.reshape(-1))
    g = g_flat.reshape(B, 3, N, KNN)

    eye = jnp.eye(KNN, dtype=jnp.float32)
    W1e = jnp.einsum("kj,cd->kcjd", eye, W1).reshape(KNN * 3 * KNN, D_MODEL)
    W1e_hi = W1e.astype(jnp.bfloat16).astype(jnp.float32)
    W1e_lo = (W1e - W1e_hi) * 512.0

    out = pl.pallas_call(
        _mlp_body,
        grid=(B, N // BM),
        in_specs=[
            pl.BlockSpec((1, BM, 3), lambda b, n: (b, n, 0)),
            pl.BlockSpec((1, 3, BM, KNN), lambda b, n: (b, 0, n, 0)),
            pl.BlockSpec((3, D_MODEL), lambda b, n: (0, 0)),
            pl.BlockSpec((KNN * 3 * KNN, D_MODEL), lambda b, n: (0, 0)),
            pl.BlockSpec((KNN * 3 * KNN, D_MODEL), lambda b, n: (0, 0)),
            pl.BlockSpec((1, D_MODEL), lambda b, n: (0, 0)),
            pl.BlockSpec((D_MODEL, D_MODEL), lambda b, n: (0, 0)),
            pl.BlockSpec((1, D_MODEL), lambda b, n: (0, 0)),
        ],
        out_specs=pl.BlockSpec((1, BM, KNN, D_MODEL), lambda b, n: (b, n, 0, 0)),
        out_shape=jax.ShapeDtypeStruct((B, N, KNN, D_MODEL), jnp.float32),
    )(pc, g, W1, W1e_hi, W1e_lo, b1r, W2, b2r)
    return out


# speedup vs baseline: 1.0979x; 1.0979x over previous
import functools

import jax
import jax.numpy as jnp
from jax import lax
from jax.experimental import pallas as pl
from jax.experimental.pallas import tpu as pltpu
from jax.experimental.pallas import tpu_sc as plsc

D_MODEL = 256
KNN = 16
N_PTS = 2048
N_BATCH = 4
BN = 512
BM = 512
N_WORKERS = 32
W_PER_B = N_WORKERS // N_BATCH
ROWS_PER_W = N_PTS // W_PER_B


def _topk_body(pc_ref, pcT_ref, idx_ref):
    n_blk = pl.program_id(1)
    q = pc_ref[0]
    qx = q[:, 0:1]
    qy = q[:, 1:2]
    qz = q[:, 2:3]
    xr = pcT_ref[0, 0:1, :]
    yr = pcT_ref[0, 1:2, :]
    zr = pcT_ref[0, 2:3, :]
    dx = qx - xr
    dy = qy - yr
    dz = qz - zr
    vals = dx * dx + dy * dy + dz * dz
    iota = lax.broadcasted_iota(jnp.int32, (BN, N_PTS), 1)
    row_ids = (
        lax.broadcasted_iota(jnp.int32, (BN, 1), 0) + n_blk * BN
    )
    vals = jnp.where(iota == row_ids, jnp.inf, vals)
    iota_i = lax.broadcasted_iota(jnp.int32, (N_PTS, 1), 0)
    iota_hi = (iota_i // 256).astype(jnp.float32)
    iota_lo = (iota_i % 256).astype(jnp.float32)
    iota_hl = jnp.concatenate([iota_hi, iota_lo], axis=1)
    cols = [row_ids.astype(jnp.float32)]
    for _ in range(KNN - 1):
        m = jnp.min(vals, axis=1, keepdims=True)
        eq = vals == m
        sel01 = jnp.where(eq, 1.0, 0.0)
        hl = jnp.dot(sel01, iota_hl, preferred_element_type=jnp.float32)
        vals = jnp.where(eq, jnp.inf, vals)
        cols.append(hl[:, 0:1] * 256.0 + hl[:, 1:2])
    idx = jnp.concatenate(cols, axis=1)
    idx_ref[0] = (idx + 0.5).astype(jnp.int32)


def _gather_body(pcT_hbm, idx_hbm, out_hbm, x_v, y_v, z_v, idx_v, gx_v, gy_v, gz_v):
    wid = lax.axis_index("s") * 2 + lax.axis_index("c")
    b = wid // W_PER_B
    r0 = (wid % W_PER_B) * ROWS_PER_W

    pltpu.sync_copy(pcT_hbm.at[pl.ds(b * 3 * N_PTS, N_PTS)], x_v)
    pltpu.sync_copy(pcT_hbm.at[pl.ds(b * 3 * N_PTS + N_PTS, N_PTS)], y_v)
    pltpu.sync_copy(pcT_hbm.at[pl.ds(b * 3 * N_PTS + 2 * N_PTS, N_PTS)], z_v)
    pltpu.sync_copy(
        idx_hbm.at[pl.ds((b * N_PTS + r0) * KNN, ROWS_PER_W * KNN)], idx_v)

    lane = lax.iota(jnp.int32, KNN)

    def body(i, carry):
        pos = lane + i * KNN
        iv = plsc.load_gather(idx_v, [pos])
        qi = jnp.full((KNN,), r0 + i, jnp.int32)
        gx = plsc.load_gather(x_v, [iv])
        gy = plsc.load_gather(y_v, [iv])
        gz = plsc.load_gather(z_v, [iv])
        qx = plsc.load_gather(x_v, [qi])
        qy = plsc.load_gather(y_v, [qi])
        qz = plsc.load_gather(z_v, [qi])
        plsc.store_scatter(gx_v, [pos], qx - gx)
        plsc.store_scatter(gy_v, [pos], qy - gy)
        plsc.store_scatter(gz_v, [pos], qz - gz)
        return carry

    lax.fori_loop(0, ROWS_PER_W, body, 0)

    base = (b * 3 * N_PTS + r0) * KNN
    pltpu.sync_copy(gx_v, out_hbm.at[pl.ds(base, ROWS_PER_W * KNN)])
    pltpu.sync_copy(gy_v, out_hbm.at[pl.ds(base + N_PTS * KNN, ROWS_PER_W * KNN)])
    pltpu.sync_copy(gz_v, out_hbm.at[pl.ds(base + 2 * N_PTS * KNN, ROWS_PER_W * KNN)])


def _mlp_body(d_ref, w1e_ref, b1_ref, w2_ref, b2_ref, out_ref):
    b1 = b1_ref[...]
    w2 = w2_ref[...]
    b2 = b2_ref[...]
    d3 = jnp.concatenate(
        [d_ref[0, 0], d_ref[0, 1], d_ref[0, 2]], axis=1)
    o0 = jnp.dot(jnp.maximum(b1, 0.0), w2, preferred_element_type=jnp.float32) + b2
    out_ref[0, :, 0, :] = jnp.broadcast_to(o0, (BM, D_MODEL))
    for k in range(1, KNN):
        h = jnp.maximum(
            jnp.dot(d3, w1e_ref[k * 48:(k + 1) * 48, :],
                    preferred_element_type=jnp.float32) + b1, 0.0)
        out_ref[0, :, k, :] = jnp.dot(h, w2, preferred_element_type=jnp.float32) + b2


def _sc_gather(pcT_flat, idx_flat):
    mesh = plsc.VectorSubcoreMesh(core_axis_name="c", subcore_axis_name="s")
    f = pl.kernel(
        _gather_body,
        out_type=jax.ShapeDtypeStruct((N_BATCH * 3 * N_PTS * KNN,), jnp.float32),
        mesh=mesh,
        compiler_params=pltpu.CompilerParams(needs_layout_passes=False),
        scratch_types=[
            pltpu.VMEM((N_PTS,), jnp.float32),
            pltpu.VMEM((N_PTS,), jnp.float32),
            pltpu.VMEM((N_PTS,), jnp.float32),
            pltpu.VMEM((ROWS_PER_W * KNN,), jnp.int32),
            pltpu.VMEM((ROWS_PER_W * KNN,), jnp.float32),
            pltpu.VMEM((ROWS_PER_W * KNN,), jnp.float32),
            pltpu.VMEM((ROWS_PER_W * KNN,), jnp.float32),
        ],
    )
    return f(pcT_flat, idx_flat)


@jax.jit
def kernel(pc, W1, b1, W2, b2):
    B, N, _ = pc.shape
    pcT = jnp.transpose(pc, (0, 2, 1))
    b1r = b1.reshape(1, D_MODEL)
    b2r = b2.reshape(1, D_MODEL)

    idx = pl.pallas_call(
        _topk_body,
        grid=(B, N // BN),
        in_specs=[
            pl.BlockSpec((1, BN, 3), lambda b, n: (b, n, 0)),
            pl.BlockSpec((1, 3, N_PTS), lambda b, n: (b, 0, 0)),
        ],
        out_specs=pl.BlockSpec((1, BN, KNN), lambda b, n: (b, n, 0)),
        out_shape=jax.ShapeDtypeStruct((B, N, KNN), jnp.int32),
    )(pc, pcT)

    d_flat = _sc_gather(pcT.reshape(-1), idx.reshape(-1))
    d = d_flat.reshape(B, 3, N, KNN)

    eye = jnp.eye(KNN, dtype=jnp.float32)
    W1e = jnp.einsum("kj,cd->kcjd", eye, W1).reshape(KNN * 3 * KNN, D_MODEL)

    out = pl.pallas_call(
        _mlp_body,
        grid=(B, N // BM),
        in_specs=[
            pl.BlockSpec((1, 3, BM, KNN), lambda b, n: (b, 0, n, 0)),
            pl.BlockSpec((KNN * 3 * KNN, D_MODEL), lambda b, n: (0, 0)),
            pl.BlockSpec((1, D_MODEL), lambda b, n: (0, 0)),
            pl.BlockSpec((D_MODEL, D_MODEL), lambda b, n: (0, 0)),
            pl.BlockSpec((1, D_MODEL), lambda b, n: (0, 0)),
        ],
        out_specs=pl.BlockSpec((1, BM, KNN, D_MODEL), lambda b, n: (b, n, 0, 0)),
        out_shape=jax.ShapeDtypeStruct((B, N, KNN, D_MODEL), jnp.float32),
    )(d, W1e, b1r, W2, b2r)
    return out
